# Initial kernel scaffold; baseline (speedup 1.0000x reference)
import jax
import jax.numpy as jnp
from jax.experimental import pallas as pl


def kernel(x_nchw, conv1_w, conv1_b, conv2_w, conv2_b, fc1_w, fc1_b, fc2_w, fc2_b):
    raise NotImplementedError("write your pallas kernel here")


# speedup vs baseline: 20.1514x; 20.1514x over previous
import functools
import math

import numpy as np

import jax
import jax.numpy as jnp
from jax.experimental import pallas as pl
from jax.experimental.pallas import tpu as pltpu

_BT = 128


def _fused_cnn_body(x_ref, t1_ref, b1_ref, t2_ref, b2_ref, w1_ref, fb1_ref,
                    w2_ref, fb2_ref, o_ref, *, bt):
    xb = x_ref[...].astype(jnp.bfloat16)
    zr = jnp.zeros((bt, 1, 28), jnp.bfloat16)
    xh = jnp.concatenate([zr, xb, zr], axis=1)

    y1 = None
    for dy in range(3):
        a = xh[:, dy:dy + 28, :].reshape(bt * 28, 28)
        d = jnp.dot(a, t1_ref[dy], preferred_element_type=jnp.float32)
        y1 = d if y1 is None else y1 + d
    pw = jnp.maximum(y1[:, :256], y1[:, 256:])
    z = pw.reshape(bt, 14, 2, 256)
    p1 = jnp.maximum(z[:, :, 0], z[:, :, 1])
    p1 = jnp.maximum(p1 + b1_ref[...].reshape(1, 1, 256), 0.0)
    p1 = p1.astype(jnp.bfloat16)

    zr2 = jnp.zeros((bt, 1, 256), jnp.bfloat16)
    p1h = jnp.concatenate([zr2, p1, zr2], axis=1)

    y2 = None
    for dy in range(3):
        a = p1h[:, dy:dy + 14, :].reshape(bt * 14, 256)
        d = jnp.dot(a, t2_ref[dy], preferred_element_type=jnp.float32)
        y2 = d if y2 is None else y2 + d
    pw2 = jnp.maximum(y2[:, :256], y2[:, 256:])
    z2 = pw2.reshape(bt, 7, 2, 256)
    p2 = jnp.maximum(z2[:, :, 0], z2[:, :, 1])
    feats = jnp.maximum(p2 + b2_ref[...].reshape(1, 1, 256), 0.0)
    feats = feats.astype(jnp.bfloat16)

    h = None
    for hh in range(7):
        d = jnp.dot(feats[:, hh, :], w1_ref[hh],
                    preferred_element_type=jnp.float32)
        h = d if h is None else h + d
    h = jnp.maximum(h + fb1_ref[...], 0.0).astype(jnp.bfloat16)

    o_ref[...] = (jnp.dot(h, w2_ref[...], preferred_element_type=jnp.float32)
                  + fb2_ref[...])


def _toeplitz1(w1):
    t = jnp.zeros((3, 28, 512), jnp.float32)
    for dy in range(3):
        for dx in range(3):
            w_out = np.arange(28)
            w_in = w_out + dx - 1
            v = (w_in >= 0) & (w_in < 28)
            wo, wi = w_out[v], w_in[v]
            cols = (wo % 2) * 256 + (wo // 2) * 16
            col_idx = cols[:, None] + np.arange(16)[None, :]
            t = t.at[dy, wi[:, None], col_idx].set(
                jnp.broadcast_to(w1[dy * 3 + dx], (len(wo), 16)))
    return t.astype(jnp.bfloat16)


def _toeplitz2(w2):
    t = jnp.zeros((3, 256, 512), jnp.float32)
    for dy in range(3):
        for dx in range(3):
            w_out = np.arange(14)
            w_in = w_out + dx - 1
            v = (w_in >= 0) & (w_in < 14)
            wo, wi = w_out[v], w_in[v]
            rows = wi[:, None] * 16 + np.arange(16)[None, :]
            cols = ((wo % 2) * 256 + (wo // 2) * 32)[:, None] + np.arange(32)[None, :]
            tap = w2[(dy * 3 + dx) * 16:(dy * 3 + dx + 1) * 16, :]
            t = t.at[dy, rows[:, :, None], cols[:, None, :]].set(
                jnp.broadcast_to(tap, (len(wo), 16, 32)))
    return t.astype(jnp.bfloat16)


def kernel(x_nchw, conv1_w, conv1_b, conv2_w, conv2_b, fc1_w, fc1_b,
           fc2_w, fc2_b):
    n = x_nchw.shape[0]
    bt = math.gcd(n, _BT)
    x = x_nchw.reshape(n, 28, 28)

    t1 = _toeplitz1(conv1_w)
    t2 = _toeplitz2(conv2_w)
    b1v = jnp.concatenate([jnp.tile(conv1_b, 14),
                           jnp.zeros((32,), jnp.float32)]).reshape(1, 256)
    b2v = jnp.concatenate([jnp.tile(conv2_b, 7),
                           jnp.zeros((32,), jnp.float32)]).reshape(1, 256)
    w1 = jnp.concatenate([fc1_w.reshape(7, 224, 128),
                          jnp.zeros((7, 32, 128), jnp.float32)],
                         axis=1).astype(jnp.bfloat16)
    w2 = fc2_w.astype(jnp.bfloat16)

    body = functools.partial(_fused_cnn_body, bt=bt)
    logits = pl.pallas_call(
        body,
        out_shape=jax.ShapeDtypeStruct((n, 128), jnp.float32),
        grid=(n // bt,),
        in_specs=[
            pl.BlockSpec((bt, 28, 28), lambda i: (i, 0, 0)),
            pl.BlockSpec((3, 28, 512), lambda i: (0, 0, 0)),
            pl.BlockSpec((1, 256), lambda i: (0, 0)),
            pl.BlockSpec((3, 256, 512), lambda i: (0, 0, 0)),
            pl.BlockSpec((1, 256), lambda i: (0, 0)),
            pl.BlockSpec((7, 256, 128), lambda i: (0, 0, 0)),
            pl.BlockSpec((1, 128), lambda i: (0, 0)),
            pl.BlockSpec((128, 128), lambda i: (0, 0)),
            pl.BlockSpec((1, 128), lambda i: (0, 0)),
        ],
        out_specs=pl.BlockSpec((bt, 128), lambda i: (i, 0)),
        compiler_params=pltpu.CompilerParams(
            dimension_semantics=("parallel",),
            vmem_limit_bytes=100 * 1024 * 1024,
        ),
    )(x, t1, b1v, t2, b2v, w1, fc1_b.reshape(1, 128), w2,
      fc2_b.reshape(1, 128))
    return logits[:, :10]


# speedup vs baseline: 44.7690x; 2.2216x over previous
import functools
import math

import numpy as np

import jax
import jax.numpy as jnp
from jax.experimental import pallas as pl
from jax.experimental.pallas import tpu as pltpu

_BT = 256


def _fused_cnn_body(x_ref, t1_ref, b1_ref, t2_ref, b2_ref, w1_ref, fb1_ref,
                    w2_ref, fb2_ref, o_ref, *, bt):
    x = x_ref[...]
    b1 = b1_ref[...]
    b2 = b2_ref[...]

    p1 = []
    for i in range(14):
        ya = jnp.dot(x[:, 64 * i:64 * i + 96], t1_ref[...],
                     preferred_element_type=jnp.float32)
        yb = jnp.dot(x[:, 64 * i + 32:64 * i + 128], t1_ref[...],
                     preferred_element_type=jnp.float32)
        m = jnp.maximum(ya, yb)
        m = jnp.maximum(m[:, :256], m[:, 256:])
        p1.append(jnp.maximum(m + b1, 0.0).astype(jnp.bfloat16))

    z256 = jnp.zeros((bt, 256), jnp.bfloat16)
    p1f = jnp.concatenate([z256] + p1 + [z256], axis=1)

    feats = []
    for i in range(7):
        ya = jnp.dot(p1f[:, 512 * i:512 * i + 768], t2_ref[...],
                     preferred_element_type=jnp.float32)
        yb = jnp.dot(p1f[:, 512 * i + 256:512 * i + 1024], t2_ref[...],
                     preferred_element_type=jnp.float32)
        m = jnp.maximum(ya, yb)
        m = jnp.maximum(m[:, :256], m[:, 256:])
        feats.append(jnp.maximum(m + b2, 0.0).astype(jnp.bfloat16))

    ff = jnp.concatenate(feats, axis=1)
    h = jnp.dot(ff, w1_ref[...], preferred_element_type=jnp.float32)
    h = jnp.maximum(h + fb1_ref[...], 0.0).astype(jnp.bfloat16)
    o_ref[...] = (jnp.dot(h, w2_ref[...], preferred_element_type=jnp.float32)
                  + fb2_ref[...])


def _toeplitz1(w1):
    t = jnp.zeros((3, 32, 512), jnp.float32)
    for dy in range(3):
        for dx in range(3):
            w_out = np.arange(28)
            w_in = w_out + dx - 1
            v = (w_in >= 0) & (w_in < 28)
            wo, wi = w_out[v], w_in[v]
            cols = (wo % 2) * 256 + (wo // 2) * 16
            col_idx = cols[:, None] + np.arange(16)[None, :]
            t = t.at[dy, wi[:, None], col_idx].set(
                jnp.broadcast_to(w1[dy * 3 + dx], (len(wo), 16)))
    return t.reshape(96, 512).astype(jnp.bfloat16)


def _toeplitz2(w2):
    t = jnp.zeros((3, 256, 512), jnp.float32)
    for dy in range(3):
        for dx in range(3):
            w_out = np.arange(14)
            w_in = w_out + dx - 1
            v = (w_in >= 0) & (w_in < 14)
            wo, wi = w_out[v], w_in[v]
            rows = wi[:, None] * 16 + np.arange(16)[None, :]
            cols = ((wo % 2) * 256 + (wo // 2) * 32)[:, None] + np.arange(32)[None, :]
            tap = w2[(dy * 3 + dx) * 16:(dy * 3 + dx + 1) * 16, :]
            t = t.at[dy, rows[:, :, None], cols[:, None, :]].set(
                jnp.broadcast_to(tap, (len(wo), 16, 32)))
    return t.reshape(768, 512).astype(jnp.bfloat16)


def kernel(x_nchw, conv1_w, conv1_b, conv2_w, conv2_b, fc1_w, fc1_b,
           fc2_w, fc2_b):
    n = x_nchw.shape[0]
    bt = math.gcd(n, _BT)

    x = jnp.pad(x_nchw.reshape(n, 28, 28), ((0, 0), (1, 1), (0, 4)))
    x = x.reshape(n, 960).astype(jnp.bfloat16)

    t1 = _toeplitz1(conv1_w)
    t2 = _toeplitz2(conv2_w)
    b1v = jnp.concatenate([jnp.tile(conv1_b, 14),
                           jnp.zeros((32,), jnp.float32)]).reshape(1, 256)
    b2v = jnp.concatenate([jnp.tile(conv2_b, 7),
                           jnp.zeros((32,), jnp.float32)]).reshape(1, 256)
    w1 = jnp.concatenate([fc1_w.reshape(7, 224, 128),
                          jnp.zeros((7, 32, 128), jnp.float32)],
                         axis=1).reshape(1792, 128).astype(jnp.bfloat16)
    w2 = fc2_w.astype(jnp.bfloat16)

    body = functools.partial(_fused_cnn_body, bt=bt)
    logits = pl.pallas_call(
        body,
        out_shape=jax.ShapeDtypeStruct((n, 128), jnp.float32),
        grid=(n // bt,),
        in_specs=[
            pl.BlockSpec((bt, 960), lambda i: (i, 0)),
            pl.BlockSpec((96, 512), lambda i: (0, 0)),
            pl.BlockSpec((1, 256), lambda i: (0, 0)),
            pl.BlockSpec((768, 512), lambda i: (0, 0)),
            pl.BlockSpec((1, 256), lambda i: (0, 0)),
            pl.BlockSpec((1792, 128), lambda i: (0, 0)),
            pl.BlockSpec((1, 128), lambda i: (0, 0)),
            pl.BlockSpec((128, 128), lambda i: (0, 0)),
            pl.BlockSpec((1, 128), lambda i: (0, 0)),
        ],
        out_specs=pl.BlockSpec((bt, 128), lambda i: (i, 0)),
        compiler_params=pltpu.CompilerParams(
            dimension_semantics=("parallel",),
            vmem_limit_bytes=100 * 1024 * 1024,
        ),
    )(x, t1, b1v, t2, b2v, w1, fc1_b.reshape(1, 128), w2,
      fc2_b.reshape(1, 128))
    return logits[:, :10]
